# baseline (speedup 1.0000x reference)
import jax
import jax.numpy as jnp
from jax.experimental import pallas as pl
from jax.experimental.pallas import tpu as pltpu

_TX = 64


def _body(coords_ref, target_ref, prior_ref, mask_ref, out_ref):
    b = pl.program_id(0)
    xt = pl.program_id(1)
    c = pl.program_id(2)
    cx = coords_ref[b, 0]
    cy = coords_ref[b, 1]
    cz = coords_ref[b, 2]

    shape = target_ref.shape[2:]
    lcx = cx - xt * _TX

    @pl.when(c == 2)
    def _():
        out_ref[0, 0] = mask_ref[0, 0]

    def masked_channel(src_ref):
        y_ids = jax.lax.broadcasted_iota(jnp.int32, (1,) + shape[1:], 1)
        z_ids = jax.lax.broadcasted_iota(jnp.int32, (1,) + shape[1:], 2)
        m_yz = (y_ids == cy) | (z_ids == cz)
        zero = jnp.zeros(shape, dtype=out_ref.dtype)
        out_ref[0, 0] = jnp.where(m_yz, src_ref[0, 0], zero)

        @pl.when((lcx >= 0) & (lcx < _TX))
        def _():
            out_ref[0, 0, pl.ds(lcx, 1)] = src_ref[0, 0, pl.ds(lcx, 1)]

    @pl.when(c == 0)
    def _():
        masked_channel(target_ref)

    @pl.when(c == 1)
    def _():
        masked_channel(prior_ref)


def kernel(full_target_img, full_prior_img, prior_mask, coords):
    B, C, W, H, D = full_target_img.shape
    nxt = W // _TX

    def in_map(b, xt, c, coords_ref):
        return (b, 0, xt, 0, 0)

    def out_map(b, xt, c, coords_ref):
        return (b, c, xt, 0, 0)

    grid_spec = pltpu.PrefetchScalarGridSpec(
        num_scalar_prefetch=1,
        grid=(B, nxt, 3),
        in_specs=[
            pl.BlockSpec((1, 1, _TX, H, D), in_map),
            pl.BlockSpec((1, 1, _TX, H, D), in_map),
            pl.BlockSpec((1, 1, _TX, H, D), in_map),
        ],
        out_specs=pl.BlockSpec((1, 1, _TX, H, D), out_map),
    )

    return pl.pallas_call(
        _body,
        grid_spec=grid_spec,
        out_shape=jax.ShapeDtypeStruct((B, 3, W, H, D), full_target_img.dtype),
    )(coords, full_target_img, full_prior_img, prior_mask)


# speedup vs baseline: 1.1750x; 1.1750x over previous
import jax
import jax.numpy as jnp
from jax.experimental import pallas as pl
from jax.experimental.pallas import tpu as pltpu

_TX = 64


def _body(coords_ref, target_ref, prior_ref, mask_ref, out_ref):
    b = pl.program_id(0)
    xt = pl.program_id(1)
    cx = coords_ref[b, 0]
    cy = coords_ref[b, 1]
    cz = coords_ref[b, 2]

    shape = target_ref.shape[2:]
    y_ids = jax.lax.broadcasted_iota(jnp.int32, (1,) + shape[1:], 1)
    z_ids = jax.lax.broadcasted_iota(jnp.int32, (1,) + shape[1:], 2)
    m_yz = (y_ids == cy) | (z_ids == cz)

    zero = jnp.zeros(shape, dtype=out_ref.dtype)
    out_ref[0, 0] = jnp.where(m_yz, target_ref[0, 0], zero)
    out_ref[0, 1] = jnp.where(m_yz, prior_ref[0, 0], zero)
    out_ref[0, 2] = mask_ref[0, 0]

    lcx = cx - xt * _TX

    @pl.when((lcx >= 0) & (lcx < _TX))
    def _():
        out_ref[0, 0, pl.ds(lcx, 1)] = target_ref[0, 0, pl.ds(lcx, 1)]
        out_ref[0, 1, pl.ds(lcx, 1)] = prior_ref[0, 0, pl.ds(lcx, 1)]


def kernel(full_target_img, full_prior_img, prior_mask, coords):
    B, C, W, H, D = full_target_img.shape
    nxt = W // _TX

    def in_map(b, xt, coords_ref):
        return (b, 0, xt, 0, 0)

    def out_map(b, xt, coords_ref):
        return (b, 0, xt, 0, 0)

    grid_spec = pltpu.PrefetchScalarGridSpec(
        num_scalar_prefetch=1,
        grid=(B, nxt),
        in_specs=[
            pl.BlockSpec((1, 1, _TX, H, D), in_map),
            pl.BlockSpec((1, 1, _TX, H, D), in_map),
            pl.BlockSpec((1, 1, _TX, H, D), in_map),
        ],
        out_specs=pl.BlockSpec((1, 3, _TX, H, D), out_map),
    )

    return pl.pallas_call(
        _body,
        grid_spec=grid_spec,
        out_shape=jax.ShapeDtypeStruct((B, 3, W, H, D), full_target_img.dtype),
    )(coords, full_target_img, full_prior_img, prior_mask)
